# baseline (speedup 1.0000x reference)
import functools

import jax
import jax.numpy as jnp
from jax import lax
from jax.experimental import pallas as pl
from jax.experimental.pallas import tpu as pltpu
from jax.experimental.pallas import tpu_sc as plsc

_MAX_NORM = 1.0
_EPS = 1e-7
_CHUNK = 128


def _rsqrt_nr(x):
    i = plsc.bitcast(x, jnp.int32)
    i = jnp.int32(0x5F3759DF) - (i >> 1)
    y = plsc.bitcast(i, jnp.float32)
    for _ in range(3):
        y = y * (1.5 - 0.5 * x * y * y)
    return y


@functools.cache
def _build(NW, NC, NCH, C, D, B):
    bpw = NCH * C
    W = 2 * D
    mesh = plsc.VectorSubcoreMesh(core_axis_name="c", subcore_axis_name="s")

    @functools.partial(
        pl.kernel,
        mesh=mesh,
        out_type=jax.ShapeDtypeStruct((B,), jnp.float32),
        scratch_types=[
            pltpu.VMEM((NCH, C), jnp.int32),
            pltpu.VMEM((NCH, C), jnp.int32),
            pltpu.VMEM((NCH, C), jnp.int32),
            pltpu.VMEM((NCH, C), jnp.int32),
            pltpu.VMEM((C, W), jnp.float32),
            pltpu.VMEM((C, W), jnp.float32),
            pltpu.VMEM((C, W), jnp.float32),
            pltpu.VMEM((C, W), jnp.float32),
            pltpu.VMEM((bpw,), jnp.float32),
            pltpu.SemaphoreType.DMA,
            pltpu.SemaphoreType.DMA,
        ],
        compiler_params=pltpu.CompilerParams(needs_layout_passes=False),
    )
    def k(urow_hbm, irow_hbm, ucol_hbm, icol_hbm, users_hbm, items_hbm,
          out_hbm, uidx, iidx, ucol, icol, ubufa, ubufb, ibufa, ibufb,
          obuf, sema, semb):
        wid = lax.axis_index("s") * NC + lax.axis_index("c")
        pltpu.sync_copy(urow_hbm.at[wid], uidx)
        pltpu.sync_copy(irow_hbm.at[wid], iidx)
        pltpu.sync_copy(ucol_hbm.at[wid], ucol)
        pltpu.sync_copy(icol_hbm.at[wid], icol)

        ubufs, ibufs, sems = [ubufa, ubufb], [ibufa, ibufb], [sema, semb]

        def fire(j):
            p = j % 2
            return [
                pltpu.async_copy(users_hbm.at[uidx.at[j]], ubufs[p], sems[p]),
                pltpu.async_copy(items_hbm.at[iidx.at[j]], ibufs[p], sems[p]),
            ]

        lanes = lax.iota(jnp.int32, 16)
        zeros = jnp.zeros((16,), jnp.float32)
        eps2 = jnp.float32(_EPS * _EPS)

        pending = fire(0)
        for j in range(NCH):
            p = j % 2
            for cp in pending:
                cp.wait()
            if j + 1 < NCH:
                pending = fire(j + 1)
            ubuf, ibuf = ubufs[p], ibufs[p]

            def group_body(g, _, j=j, ubuf=ubuf, ibuf=ibuf):
                rows = g * 16 + lanes
                ucb = ucol[j, pl.ds(g * 16, 16)]
                icb = icol[j, pl.ds(g * 16, 16)]

                def d_body(d, carry):
                    acc, nu, nv = carry
                    u = plsc.load_gather(ubuf, [rows, ucb + d])
                    v = plsc.load_gather(ibuf, [rows, icb + d])
                    return acc + u * v, nu + u * u, nv + v * v

                acc, nu, nv = lax.fori_loop(
                    0, D, d_body, (zeros, zeros, zeros))
                su = jnp.minimum(jnp.float32(_MAX_NORM),
                                 _rsqrt_nr(jnp.maximum(nu, eps2)))
                sv = jnp.minimum(jnp.float32(_MAX_NORM),
                                 _rsqrt_nr(jnp.maximum(nv, eps2)))
                x = acc * su * sv
                obuf[pl.ds(j * C + g * 16, 16)] = 1.0 / (1.0 + jnp.exp(-x))
                return 0

            lax.fori_loop(0, C // 16, group_body, 0)

        pltpu.sync_copy(obuf, out_hbm.at[pl.ds(wid * bpw, bpw)])

    return k


@jax.jit
def kernel(user_ids, item_ids, users, items):
    B = user_ids.shape[0]
    D = users.shape[1]
    info = plsc.get_sparse_core_info()
    NC, NS = info.num_cores, info.num_subcores
    NW = NC * NS
    NCH = B // (NW * _CHUNK)
    uid = user_ids.astype(jnp.int32)
    iid = item_ids.astype(jnp.int32)
    urow = (uid >> 1).reshape(NW, NCH, _CHUNK)
    irow = (iid >> 1).reshape(NW, NCH, _CHUNK)
    ucol = ((uid & 1) * D).reshape(NW, NCH, _CHUNK)
    icol = ((iid & 1) * D).reshape(NW, NCH, _CHUNK)
    users2 = users.reshape(users.shape[0] // 2, 2 * D)
    items2 = items.reshape(items.shape[0] // 2, 2 * D)
    k = _build(NW, NC, NCH, _CHUNK, D, B)
    return k(urow, irow, ucol, icol, users2, items2)
